# baseline (speedup 1.0000x reference)
import functools
import math

import jax
import jax.numpy as jnp
from jax import lax
from jax.experimental import pallas as pl
from jax.experimental.pallas import tpu as pltpu
from jax.experimental.pallas import tpu_sc as plsc

_SMOOTHING = 0.1
_CONFIDENCE = 1.0 - _SMOOTHING
_N = 1024
_V = 100000
_EPS = _SMOOTHING / (_V - 1)
_CONST = _N * ((_V - 1) * _EPS * math.log(_EPS) + _CONFIDENCE * math.log(_CONFIDENCE))

_NW = 32
_PER = _N // _NW
_L = 16

_RB = 32
_NB = _N // _RB


def _sc_gather_body(x1_hbm, tgt_hbm, out_hbm, tv, iv, vals, vbuf, sem):
    wid = lax.axis_index("s") * 2 + lax.axis_index("c")
    base = wid * _PER
    pltpu.sync_copy(tgt_hbm.at[pl.ds(base, _PER)], tv)
    for j in range(_PER // _L):
        t16 = tv[pl.ds(j * _L, _L)]
        rows = (base + j * _L) + lax.iota(jnp.int32, _L)
        iv[pl.ds(j * _L, _L)] = rows * _V + t16
    pltpu.async_copy(x1_hbm.at[iv], vals, sem).wait()
    acc = vals[pl.ds(0, _L)]
    for j in range(1, _PER // _L):
        acc = acc + vals[pl.ds(j * _L, _L)]
    vbuf[...] = acc
    pltpu.sync_copy(vbuf, out_hbm.at[wid])


_sc_gather = functools.partial(
    pl.kernel,
    mesh=plsc.VectorSubcoreMesh(core_axis_name="c", subcore_axis_name="s"),
    out_type=jax.ShapeDtypeStruct((_NW, _L), jnp.float32),
    scratch_types=[
        pltpu.VMEM((_PER,), jnp.int32),
        pltpu.VMEM((_PER,), jnp.int32),
        pltpu.VMEM((_PER,), jnp.float32),
        pltpu.VMEM((_L,), jnp.float32),
        pltpu.SemaphoreType.DMA,
    ],
)(_sc_gather_body)


def _tc_body(g_ref, x_ref, out_ref):
    b = pl.program_id(0)

    @pl.when(b == 0)
    def _init():
        g = g_ref[...]
        init = jnp.float32(_CONST) - jnp.float32(_CONFIDENCE - _EPS) * jnp.sum(g)
        out_ref[...] = init.reshape(1, 1)

    xb = x_ref[...]
    out_ref[...] -= (jnp.float32(_EPS) * jnp.sum(xb)).reshape(1, 1)


def kernel(x, target):
    tgt = target.astype(jnp.int32)
    x1 = x.reshape(-1)
    g = _sc_gather(x1, tgt)
    out = pl.pallas_call(
        _tc_body,
        grid=(_NB,),
        in_specs=[
            pl.BlockSpec((_NW, _L), lambda b: (0, 0)),
            pl.BlockSpec((_RB, _V), lambda b: (b, 0)),
        ],
        out_specs=pl.BlockSpec((1, 1), lambda b: (0, 0)),
        out_shape=jax.ShapeDtypeStruct((1, 1), jnp.float32),
        compiler_params=pltpu.CompilerParams(
            dimension_semantics=("arbitrary",),
        ),
    )(g, x)
    return out[0, 0]


# speedup vs baseline: 2.2035x; 2.2035x over previous
import math

import jax
import jax.numpy as jnp
from jax import lax
from jax.experimental import pallas as pl
from jax.experimental.pallas import tpu as pltpu

_SMOOTHING = 0.1
_CONFIDENCE = 1.0 - _SMOOTHING
_N = 1024
_V = 100000
_EPS = _SMOOTHING / (_V - 1)
_CONST = _N * ((_V - 1) * _EPS * math.log(_EPS) + _CONFIDENCE * math.log(_CONFIDENCE))

_RB = 32
_NB = _N // _RB


def _body(tgt_ref, x_ref, out_ref):
    b = pl.program_id(0)

    @pl.when(b == 0)
    def _init():
        out_ref[...] = jnp.full((1, 1), _CONST, jnp.float32)

    xb = x_ref[...]
    tgt = tgt_ref[0, 0, :]
    col = lax.broadcasted_iota(jnp.int32, (_RB, _V), 1)
    is_target = col == tgt[:, None]
    w = jnp.where(is_target, jnp.float32(_CONFIDENCE), jnp.float32(_EPS))
    out_ref[...] -= jnp.sum(xb * w).reshape(1, 1)


def kernel(x, target):
    tgt = target.astype(jnp.int32).reshape(_NB, 1, _RB)
    out = pl.pallas_call(
        _body,
        grid=(_NB,),
        in_specs=[
            pl.BlockSpec((1, 1, _RB), lambda b: (b, 0, 0)),
            pl.BlockSpec((_RB, _V), lambda b: (b, 0)),
        ],
        out_specs=pl.BlockSpec((1, 1), lambda b: (0, 0)),
        out_shape=jax.ShapeDtypeStruct((1, 1), jnp.float32),
        compiler_params=pltpu.CompilerParams(
            dimension_semantics=("arbitrary",),
        ),
    )(tgt, x)
    return out[0, 0]
